# baseline (speedup 1.0000x reference)
import functools

import jax
import jax.numpy as jnp
from jax.experimental import pallas as pl
from jax.experimental.pallas import tpu as pltpu

DIM = 768
E = 8
K = 2
DFF = DIM * 4
EPS = 1e-5
N_TOK = 2048
BT = 256
GMAX = 23


def _routing_body(x_ref, Wg_ref, wr_ref, meta_ref, xb_ref):
    logits = jnp.dot(x_ref[...], Wg_ref[...],
                     preferred_element_type=jnp.float32)
    eidx = jax.lax.broadcasted_iota(jnp.int32, (N_TOK, E), 1)
    m1 = jnp.max(logits, axis=1, keepdims=True)
    i1 = jnp.argmax(logits, axis=1).reshape(N_TOK, 1)
    masked = jnp.where(eidx == i1, -jnp.inf, logits)
    i2 = jnp.argmax(masked, axis=1).reshape(N_TOK, 1)
    m2 = jnp.max(masked, axis=1, keepdims=True)
    z = jnp.exp(m2 - m1)
    w1 = 1.0 / (1.0 + z)
    w2 = z / (1.0 + z)
    wfull = jnp.where(eidx == i1, w1, jnp.where(eidx == i2, w2, 0.0))
    maskf = jnp.where(eidx == i1, 1.0, jnp.where(eidx == i2, 1.0, 0.0))

    r_io = jax.lax.broadcasted_iota(jnp.int32, (N_TOK, N_TOK), 0)
    c_io = jax.lax.broadcasted_iota(jnp.int32, (N_TOK, N_TOK), 1)
    Lstrict = (c_io < r_io).astype(jnp.bfloat16)
    rank = jnp.dot(Lstrict, maskf.astype(jnp.bfloat16),
                   preferred_element_type=jnp.float32)
    rankT = rank.T
    wT = wfull.T
    maskT = maskf.T.astype(jnp.int32)
    R = jnp.where(maskT == 1, rankT, -1.0)
    counts = jnp.sum(maskT, axis=1, keepdims=True)

    num_tiles = (counts + (BT - 1)) // BT
    tri_r = jax.lax.broadcasted_iota(jnp.int32, (E, E), 0)
    tri_c = jax.lax.broadcasted_iota(jnp.int32, (E, E), 1)
    tri8 = (tri_c <= tri_r).astype(jnp.float32)
    cumT = jnp.dot(tri8, num_tiles.astype(jnp.float32),
                   preferred_element_type=jnp.float32).astype(jnp.int32)
    cumT_ex = cumT - num_tiles
    g_iota = jax.lax.broadcasted_iota(jnp.int32, (E, GMAX), 1)
    te = jnp.sum((g_iota >= cumT).astype(jnp.int32), axis=0, keepdims=True)
    te = jnp.minimum(te, E - 1)
    e_iota = jax.lax.broadcasted_iota(jnp.int32, (E, GMAX), 0)
    cumT_ex_g = jnp.sum(jnp.where(e_iota == te, cumT_ex, 0),
                        axis=0, keepdims=True)
    g_row = jax.lax.broadcasted_iota(jnp.int32, (1, GMAX), 1)
    p0 = (g_row - cumT_ex_g) * BT
    real = (g_row < cumT[E - 1:E, :]).astype(jnp.int32)

    wr_ref[:, 0:1, :] = wT.reshape(E, 1, N_TOK)
    wr_ref[:, 1:2, :] = R.reshape(E, 1, N_TOK)
    meta_ref[0:1, :] = te
    meta_ref[1:2, :] = p0
    meta_ref[2:3, :] = real
    xb_ref[...] = x_ref[...].astype(jnp.bfloat16)


def _ffn_body(sp_ref, xb_ref, wr_ref, W1_ref, aff_ref, W2_ref, out_ref):
    g = pl.program_id(0)
    p0 = sp_ref[GMAX + g]
    real = sp_ref[2 * GMAX + g]

    @pl.when(g == 0)
    def _init():
        out_ref[...] = jnp.zeros_like(out_ref)

    @pl.when(real == 1)
    def _compute():
        Rb = jnp.broadcast_to(wr_ref[0, 1:2, :], (BT, N_TOK))
        target = (jax.lax.broadcasted_iota(jnp.int32, (BT, N_TOK), 0)
                  + p0).astype(jnp.float32)
        G = (Rb == target).astype(jnp.bfloat16)
        Gw = G * jnp.broadcast_to(wr_ref[0, 0:1, :],
                                  (BT, N_TOK)).astype(jnp.bfloat16)
        xg = jnp.dot(G, xb_ref[...],
                     preferred_element_type=jnp.float32)
        h = jnp.dot(xg, W1_ref[0],
                    preferred_element_type=jnp.float32) + aff_ref[0, 0]
        h = h * jax.nn.sigmoid(h)
        mu = jnp.mean(h, axis=1, keepdims=True)
        hc = h - mu
        var = jnp.mean(hc * hc, axis=1, keepdims=True)
        h = hc * jax.lax.rsqrt(var + EPS) * aff_ref[0, 1] + aff_ref[0, 2]
        y = jnp.dot(h, W2_ref[0],
                    preferred_element_type=jnp.float32) + aff_ref[0, 3, 0:DIM]
        contrib = jax.lax.dot_general(
            Gw, y.astype(jnp.bfloat16),
            dimension_numbers=(((0,), (0,)), ((), ())),
            preferred_element_type=jnp.float32)
        out_ref[...] += contrib


@jax.jit
def kernel(x, Wg, W1, b1, ln_g, ln_b, W2, b2):
    wr, meta, xb = pl.pallas_call(
        _routing_body,
        out_shape=[
            jax.ShapeDtypeStruct((E, 2, N_TOK), jnp.float32),
            jax.ShapeDtypeStruct((3, GMAX), jnp.int32),
            jax.ShapeDtypeStruct((N_TOK, DIM), jnp.bfloat16),
        ],
    )(x, Wg)
    sp = meta.reshape(3 * GMAX)
    aff = jnp.concatenate(
        [b1[:, None, :], ln_g[:, None, :], ln_b[:, None, :],
         jnp.pad(b2, ((0, 0), (0, DFF - DIM)))[:, None, :]], axis=1)

    grid_spec = pltpu.PrefetchScalarGridSpec(
        num_scalar_prefetch=1,
        grid=(GMAX,),
        in_specs=[
            pl.BlockSpec((N_TOK, DIM), lambda g, sp: (0, 0)),
            pl.BlockSpec((1, 2, N_TOK), lambda g, sp: (sp[g], 0, 0)),
            pl.BlockSpec((1, DIM, DFF), lambda g, sp: (sp[g], 0, 0)),
            pl.BlockSpec((1, 4, DFF), lambda g, sp: (sp[g], 0, 0)),
            pl.BlockSpec((1, DFF, DIM), lambda g, sp: (sp[g], 0, 0)),
        ],
        out_specs=pl.BlockSpec((N_TOK, DIM), lambda g, sp: (0, 0)),
    )
    out = pl.pallas_call(
        _ffn_body,
        grid_spec=grid_spec,
        out_shape=jax.ShapeDtypeStruct((N_TOK, DIM), jnp.float32),
        compiler_params=pltpu.CompilerParams(
            dimension_semantics=("arbitrary",),
        ),
    )(sp, xb, wr, W1, aff, W2)
    return out


# speedup vs baseline: 1.0030x; 1.0030x over previous
import jax
import jax.numpy as jnp
from jax.experimental import pallas as pl
from jax.experimental.pallas import tpu as pltpu

DIM = 768
E = 8
K = 2
DFF = DIM * 4
EPS = 1e-5
N_TOK = 2048
BT = 256
GMAX = 23


def _routing_body(x_ref, Wg_ref, wr_ref, meta_ref, xb_ref):
    logits = jnp.dot(x_ref[...], Wg_ref[...],
                     preferred_element_type=jnp.float32)
    eidx = jax.lax.broadcasted_iota(jnp.int32, (N_TOK, E), 1)
    m1 = jnp.max(logits, axis=1, keepdims=True)
    i1 = jnp.argmax(logits, axis=1).reshape(N_TOK, 1)
    masked = jnp.where(eidx == i1, -jnp.inf, logits)
    i2 = jnp.argmax(masked, axis=1).reshape(N_TOK, 1)
    m2 = jnp.max(masked, axis=1, keepdims=True)
    z = jnp.exp(m2 - m1)
    w1 = 1.0 / (1.0 + z)
    w2 = z / (1.0 + z)
    wfull = jnp.where(eidx == i1, w1, jnp.where(eidx == i2, w2, 0.0))
    maskf = jnp.where(eidx == i1, 1.0, jnp.where(eidx == i2, 1.0, 0.0))

    r_io = jax.lax.broadcasted_iota(jnp.int32, (N_TOK, N_TOK), 0)
    c_io = jax.lax.broadcasted_iota(jnp.int32, (N_TOK, N_TOK), 1)
    Lstrict = (c_io < r_io).astype(jnp.bfloat16)
    rank = jnp.dot(Lstrict, maskf.astype(jnp.bfloat16),
                   preferred_element_type=jnp.float32)
    rankT = rank.T
    wT = wfull.T
    maskT = maskf.T.astype(jnp.int32)
    R = jnp.where(maskT == 1, rankT, -1.0)
    counts = jnp.sum(maskT, axis=1, keepdims=True)

    num_tiles = (counts + (BT - 1)) // BT
    tri_r = jax.lax.broadcasted_iota(jnp.int32, (E, E), 0)
    tri_c = jax.lax.broadcasted_iota(jnp.int32, (E, E), 1)
    tri8 = (tri_c <= tri_r).astype(jnp.float32)
    cumT = jnp.dot(tri8, num_tiles.astype(jnp.float32),
                   preferred_element_type=jnp.float32).astype(jnp.int32)
    cumT_ex = cumT - num_tiles
    g_iota = jax.lax.broadcasted_iota(jnp.int32, (E, GMAX), 1)
    te = jnp.sum((g_iota >= cumT).astype(jnp.int32), axis=0, keepdims=True)
    te = jnp.minimum(te, E - 1)
    e_iota = jax.lax.broadcasted_iota(jnp.int32, (E, GMAX), 0)
    cumT_ex_g = jnp.sum(jnp.where(e_iota == te, cumT_ex, 0),
                        axis=0, keepdims=True)
    g_row = jax.lax.broadcasted_iota(jnp.int32, (1, GMAX), 1)
    p0 = (g_row - cumT_ex_g) * BT
    real = (g_row < cumT[E - 1:E, :]).astype(jnp.int32)

    wr_ref[:, 0:1, :] = wT.reshape(E, 1, N_TOK)
    wr_ref[:, 1:2, :] = R.reshape(E, 1, N_TOK)
    meta_ref[0:1, :] = te
    meta_ref[1:2, :] = p0
    meta_ref[2:3, :] = real
    xb_ref[...] = x_ref[...].astype(jnp.bfloat16)


def _ffn_body(sp_ref, xb_ref, wr_ref, W1_ref, aff_ref, W2_ref, out_ref):
    g = pl.program_id(0)
    p0 = sp_ref[GMAX + g]
    real = sp_ref[2 * GMAX + g]

    @pl.when(g == 0)
    def _init():
        out_ref[...] = jnp.zeros_like(out_ref)

    @pl.when(real == 1)
    def _compute():
        Rb = jnp.broadcast_to(wr_ref[0, 1:2, :], (BT, N_TOK))
        target = (jax.lax.broadcasted_iota(jnp.int32, (BT, N_TOK), 0)
                  + p0).astype(jnp.float32)
        G = (Rb == target).astype(jnp.bfloat16)
        Gw = G * jnp.broadcast_to(wr_ref[0, 0:1, :],
                                  (BT, N_TOK)).astype(jnp.bfloat16)
        xg = jnp.dot(G, xb_ref[...],
                     preferred_element_type=jnp.float32)
        h = jnp.dot(xg, W1_ref[0],
                    preferred_element_type=jnp.float32) + aff_ref[0, 0]
        h = h * jax.nn.sigmoid(h)
        mu = jnp.mean(h, axis=1, keepdims=True)
        hc = h - mu
        var = jnp.mean(hc * hc, axis=1, keepdims=True)
        h = hc * jax.lax.rsqrt(var + EPS) * aff_ref[0, 1] + aff_ref[0, 2]
        y = jnp.dot(h, W2_ref[0],
                    preferred_element_type=jnp.float32) + aff_ref[0, 3, 0:DIM]
        contrib = jax.lax.dot_general(
            Gw, y.astype(jnp.bfloat16),
            dimension_numbers=(((0,), (0,)), ((), ())),
            preferred_element_type=jnp.float32)
        out_ref[...] += contrib


@jax.jit
def kernel(x, Wg, W1, b1, ln_g, ln_b, W2, b2):
    wr, meta, xb = pl.pallas_call(
        _routing_body,
        out_shape=[
            jax.ShapeDtypeStruct((E, 2, N_TOK), jnp.float32),
            jax.ShapeDtypeStruct((3, GMAX), jnp.int32),
            jax.ShapeDtypeStruct((N_TOK, DIM), jnp.bfloat16),
        ],
    )(x, Wg)
    sp = meta.reshape(3 * GMAX)
    aff = jnp.concatenate(
        [b1[:, None, :], ln_g[:, None, :], ln_b[:, None, :],
         jnp.pad(b2, ((0, 0), (0, DFF - DIM)))[:, None, :]], axis=1)

    grid_spec = pltpu.PrefetchScalarGridSpec(
        num_scalar_prefetch=1,
        grid=(GMAX,),
        in_specs=[
            pl.BlockSpec((N_TOK, DIM), lambda g, sp: (0, 0)),
            pl.BlockSpec((1, 2, N_TOK), lambda g, sp: (sp[g], 0, 0)),
            pl.BlockSpec((1, DIM, DFF), lambda g, sp: (sp[g], 0, 0)),
            pl.BlockSpec((1, 4, DFF), lambda g, sp: (sp[g], 0, 0)),
            pl.BlockSpec((1, DFF, DIM), lambda g, sp: (sp[g], 0, 0)),
        ],
        out_specs=pl.BlockSpec((N_TOK, DIM), lambda g, sp: (0, 0)),
    )
    out = pl.pallas_call(
        _ffn_body,
        grid_spec=grid_spec,
        out_shape=jax.ShapeDtypeStruct((N_TOK, DIM), jnp.float32),
        compiler_params=pltpu.CompilerParams(
            dimension_semantics=("arbitrary",),
        ),
    )(sp, xb, wr, W1, aff, W2)
    return out


# speedup vs baseline: 1.0043x; 1.0013x over previous
import jax
import jax.numpy as jnp
from jax.experimental import pallas as pl
from jax.experimental.pallas import tpu as pltpu

DIM = 768
E = 8
K = 2
DFF = DIM * 4
EPS = 1e-5
N_TOK = 2048
BT = 256
GMAX = 23


def _routing_body(x_ref, Wg_ref, wr_ref, meta_ref, xb_ref):
    logits = jnp.dot(x_ref[...], Wg_ref[...],
                     preferred_element_type=jnp.float32)
    eidx = jax.lax.broadcasted_iota(jnp.int32, (N_TOK, E), 1)
    m1 = jnp.max(logits, axis=1, keepdims=True)
    i1 = jnp.argmax(logits, axis=1).reshape(N_TOK, 1)
    masked = jnp.where(eidx == i1, -jnp.inf, logits)
    i2 = jnp.argmax(masked, axis=1).reshape(N_TOK, 1)
    m2 = jnp.max(masked, axis=1, keepdims=True)
    z = jnp.exp(m2 - m1)
    w1 = 1.0 / (1.0 + z)
    w2 = z / (1.0 + z)
    wfull = jnp.where(eidx == i1, w1, jnp.where(eidx == i2, w2, 0.0))
    maskf = jnp.where(eidx == i1, 1.0, jnp.where(eidx == i2, 1.0, 0.0))

    r_io = jax.lax.broadcasted_iota(jnp.int32, (N_TOK, N_TOK), 0)
    c_io = jax.lax.broadcasted_iota(jnp.int32, (N_TOK, N_TOK), 1)
    Lstrict = (c_io < r_io).astype(jnp.bfloat16)
    rank = jnp.dot(Lstrict, maskf.astype(jnp.bfloat16),
                   preferred_element_type=jnp.float32)
    rankT = rank.T
    wT = wfull.T
    maskT = maskf.T.astype(jnp.int32)
    R = jnp.where(maskT == 1, rankT, -1.0)
    counts = jnp.sum(maskT, axis=1, keepdims=True)

    num_tiles = (counts + (BT - 1)) // BT
    tri_r = jax.lax.broadcasted_iota(jnp.int32, (E, E), 0)
    tri_c = jax.lax.broadcasted_iota(jnp.int32, (E, E), 1)
    tri8 = (tri_c <= tri_r).astype(jnp.float32)
    cumT = jnp.dot(tri8, num_tiles.astype(jnp.float32),
                   preferred_element_type=jnp.float32).astype(jnp.int32)
    cumT_ex = cumT - num_tiles
    g_iota = jax.lax.broadcasted_iota(jnp.int32, (E, GMAX), 1)
    te = jnp.sum((g_iota >= cumT).astype(jnp.int32), axis=0, keepdims=True)
    te = jnp.minimum(te, E - 1)
    e_iota = jax.lax.broadcasted_iota(jnp.int32, (E, GMAX), 0)
    cumT_ex_g = jnp.sum(jnp.where(e_iota == te, cumT_ex, 0),
                        axis=0, keepdims=True)
    g_row = jax.lax.broadcasted_iota(jnp.int32, (1, GMAX), 1)
    p0 = (g_row - cumT_ex_g) * BT
    real = (g_row < cumT[E - 1:E, :]).astype(jnp.int32)

    wr_ref[:, 0:1, :] = wT.reshape(E, 1, N_TOK)
    wr_ref[:, 1:2, :] = R.reshape(E, 1, N_TOK)
    meta_ref[0:1, :] = te
    meta_ref[1:2, :] = p0
    meta_ref[2:3, :] = real
    xb_ref[...] = x_ref[...].astype(jnp.bfloat16)


def _ffn_body(sp_ref, xb_ref, wr_ref, W1_ref, aff_ref, W2_ref, out_ref):
    g = pl.program_id(0)
    p0 = sp_ref[GMAX + g]
    real = sp_ref[2 * GMAX + g]

    @pl.when(g == 0)
    def _init():
        out_ref[...] = jnp.zeros_like(out_ref)

    @pl.when(real == 1)
    def _compute():
        Rb = jnp.broadcast_to(wr_ref[0, 1:2, :], (BT, N_TOK))
        target = (jax.lax.broadcasted_iota(jnp.int32, (BT, N_TOK), 0)
                  + p0).astype(jnp.float32)
        G = (Rb == target).astype(jnp.bfloat16)
        Gw = G * jnp.broadcast_to(wr_ref[0, 0:1, :], (BT, N_TOK))
        xg = jnp.dot(G, xb_ref[...],
                     preferred_element_type=jnp.float32)
        h = jnp.dot(xg, W1_ref[0],
                    preferred_element_type=jnp.float32) + aff_ref[0, 0]
        h = h * jax.nn.sigmoid(h)
        mu = jnp.mean(h, axis=1, keepdims=True)
        hc = h - mu
        var = jnp.mean(hc * hc, axis=1, keepdims=True)
        h = hc * jax.lax.rsqrt(var + EPS) * aff_ref[0, 1] + aff_ref[0, 2]
        y = jnp.dot(h, W2_ref[0],
                    preferred_element_type=jnp.float32) + aff_ref[0, 3, 0:DIM]
        contrib = jax.lax.dot_general(
            Gw, y,
            dimension_numbers=(((0,), (0,)), ((), ())),
            preferred_element_type=jnp.float32)
        out_ref[...] += contrib


@jax.jit
def kernel(x, Wg, W1, b1, ln_g, ln_b, W2, b2):
    wr, meta, xb = pl.pallas_call(
        _routing_body,
        out_shape=[
            jax.ShapeDtypeStruct((E, 2, N_TOK), jnp.float32),
            jax.ShapeDtypeStruct((3, GMAX), jnp.int32),
            jax.ShapeDtypeStruct((N_TOK, DIM), jnp.bfloat16),
        ],
    )(x, Wg)
    sp = meta.reshape(3 * GMAX)
    aff = jnp.concatenate(
        [b1[:, None, :], ln_g[:, None, :], ln_b[:, None, :],
         jnp.pad(b2, ((0, 0), (0, DFF - DIM)))[:, None, :]], axis=1)

    grid_spec = pltpu.PrefetchScalarGridSpec(
        num_scalar_prefetch=1,
        grid=(GMAX,),
        in_specs=[
            pl.BlockSpec((N_TOK, DIM), lambda g, sp: (0, 0)),
            pl.BlockSpec((1, 2, N_TOK), lambda g, sp: (sp[g], 0, 0)),
            pl.BlockSpec((1, DIM, DFF), lambda g, sp: (sp[g], 0, 0)),
            pl.BlockSpec((1, 4, DFF), lambda g, sp: (sp[g], 0, 0)),
            pl.BlockSpec((1, DFF, DIM), lambda g, sp: (sp[g], 0, 0)),
        ],
        out_specs=pl.BlockSpec((N_TOK, DIM), lambda g, sp: (0, 0)),
    )
    out = pl.pallas_call(
        _ffn_body,
        grid_spec=grid_spec,
        out_shape=jax.ShapeDtypeStruct((N_TOK, DIM), jnp.float32),
        compiler_params=pltpu.CompilerParams(
            dimension_semantics=("arbitrary",),
        ),
    )(sp, xb, wr, W1, aff, W2)
    return out
